# Initial kernel scaffold; baseline (speedup 1.0000x reference)
import jax
import jax.numpy as jnp
from jax.experimental import pallas as pl


def kernel(x, edge_index, edge_attr, W_e, b_e, W1, b1, W2, b2):
    raise NotImplementedError("write your pallas kernel here")


# speedup vs baseline: 2.5765x; 2.5765x over previous
import functools

import jax
import jax.numpy as jnp
from jax import lax
from jax.experimental import pallas as pl
from jax.experimental.pallas import tpu as pltpu
from jax.experimental.pallas import tpu_sc as plsc

N = 10000
N_PAD = 10240
E = 320000
D = 128
NC = 2
NS = 16
NW = NC * NS
EPT = E // NW
K = 80
ZC = 128

def _build_sc_kernel(n, n_pad, e_total, d, k, zc, interpret=False):
    ept = e_total // NW
    nchunk = ept // k
    rpt = n_pad // NS
    vpr = d // 16
    mesh = plsc.VectorSubcoreMesh(
        core_axis_name="c", subcore_axis_name="s",
        num_cores=NC, num_subcores=NS)

    @functools.partial(
        pl.kernel,
        out_type=jax.ShapeDtypeStruct((NC, n_pad, d), jnp.float32),
        mesh=mesh,
        interpret=interpret,
        scratch_types=[
            pltpu.VMEM((k, d), jnp.float32),
            pltpu.VMEM((k, d), jnp.float32),
            pltpu.VMEM((k,), jnp.int32),
            pltpu.VMEM((k,), jnp.int32),
            pltpu.VMEM((zc, d), jnp.float32),
            pltpu.VMEM_SHARED((n_pad, d), jnp.float32),
            pltpu.SemaphoreType.DMA,
        ],
    )
    def _sc_gather_scatter(h_hbm, e_hbm, src_hbm, dst_hbm, out_hbm,
                           ebuf, hbuf, sidx, didx, zbuf, agg, sem):
        c = lax.axis_index("c")
        s = lax.axis_index("s")
        tile_base = (c * NS + s) * ept

        def _zfill(r, _):
            for cc in range(vpr):
                zbuf[r, pl.ds(cc * 16, 16)] = jnp.zeros((16,), jnp.float32)
            return 0
        lax.fori_loop(0, zc, _zfill, 0, unroll=False)
        row0 = s * rpt

        def _zcopy(t, _):
            pltpu.sync_copy(zbuf, agg.at[pl.ds(row0 + t * zc, zc)])
            return 0
        lax.fori_loop(0, rpt // zc, _zcopy, 0, unroll=False)
        plsc.subcore_barrier()

        def _chunk(j, _):
            base = tile_base + j * k
            pltpu.sync_copy(src_hbm.at[pl.ds(base, k)], sidx)
            pltpu.sync_copy(e_hbm.at[pl.ds(base, k)], ebuf)
            pltpu.async_copy(h_hbm.at[sidx], hbuf, sem).wait()

            def _relu_row(r, _):
                for cc in range(vpr):
                    sl = pl.ds(cc * 16, 16)
                    ebuf[r, sl] = jnp.maximum(ebuf[r, sl] + hbuf[r, sl], 0.0)
                return 0
            lax.fori_loop(0, k, _relu_row, 0, unroll=False)

            pltpu.sync_copy(dst_hbm.at[pl.ds(base, k)], didx)
            pltpu.sync_copy(ebuf, agg.at[didx], add=True)
            return 0
        lax.fori_loop(0, nchunk, _chunk, 0, unroll=False)

        plsc.subcore_barrier()

        def _drain(t, _):
            pltpu.sync_copy(agg.at[pl.ds(row0 + t * zc, zc)],
                            out_hbm.at[c, pl.ds(row0 + t * zc, zc)])
            return 0
        lax.fori_loop(0, rpt // zc, _drain, 0, unroll=False)

    return _sc_gather_scatter


_SC_CACHE = {}


def _sc_gather_scatter(h, e, src, dst):
    if "k" not in _SC_CACHE:
        _SC_CACHE["k"] = _build_sc_kernel(N, N_PAD, E, D, K, ZC)
    return _SC_CACHE["k"](h, e, src, dst)


_BE = 8000


def _eproj_body(ea_ref, we_ref, be_ref, out_ref):
    out_ref[...] = jnp.dot(
        ea_ref[...], we_ref[...],
        preferred_element_type=jnp.float32,
        precision=lax.Precision.HIGHEST) + be_ref[...]


def _eproj(edge_attr, W_e, b_e):
    ed = edge_attr.shape[1]
    return pl.pallas_call(
        _eproj_body,
        grid=(E // _BE,),
        in_specs=[
            pl.BlockSpec((_BE, ed), lambda i: (i, 0)),
            pl.BlockSpec((ed, D), lambda i: (0, 0)),
            pl.BlockSpec((1, D), lambda i: (0, 0)),
        ],
        out_specs=pl.BlockSpec((_BE, D), lambda i: (i, 0)),
        out_shape=jax.ShapeDtypeStruct((E, D), jnp.float32),
    )(edge_attr, W_e, b_e.reshape(1, D))


_BN = 2000


def _mlp_body(p0_ref, p1_ref, h_ref, w1_ref, b1_ref, w2_ref, b2_ref, out_ref):
    t = p0_ref[0] + p1_ref[0] + h_ref[...]
    t = jnp.maximum(
        jnp.dot(t, w1_ref[...], preferred_element_type=jnp.float32,
                precision=lax.Precision.HIGHEST) + b1_ref[...], 0.0)
    t = jnp.dot(t, w2_ref[...], preferred_element_type=jnp.float32,
                precision=lax.Precision.HIGHEST) + b2_ref[...]
    out_ref[...] = jnp.maximum(t, 0.0)


def _mlp(p, h, W1, b1, W2, b2):
    nb = N // _BN
    return pl.pallas_call(
        _mlp_body,
        grid=(nb,),
        in_specs=[
            pl.BlockSpec((1, _BN, D), lambda i: (0, i, 0)),
            pl.BlockSpec((1, _BN, D), lambda i: (1, i, 0)),
            pl.BlockSpec((_BN, D), lambda i: (i, 0)),
            pl.BlockSpec((D, D), lambda i: (0, 0)),
            pl.BlockSpec((1, D), lambda i: (0, 0)),
            pl.BlockSpec((D, D), lambda i: (0, 0)),
            pl.BlockSpec((1, D), lambda i: (0, 0)),
        ],
        out_specs=pl.BlockSpec((_BN, D), lambda i: (i, 0)),
        out_shape=jax.ShapeDtypeStruct((N, D), jnp.float32),
    )(p, p, h, W1, b1.reshape(1, D), W2, b2.reshape(1, D))


NUM_LAYERS = 2


def kernel(x, edge_index, edge_attr, W_e, b_e, W1, b1, W2, b2):
    src = edge_index[0]
    dst = edge_index[1]
    e = _eproj(edge_attr, W_e, b_e)
    h = x
    for _ in range(NUM_LAYERS):
        p = _sc_gather_scatter(h, e, src, dst)
        h = _mlp(p, h, W1, b1, W2, b2)
    return h
